# baseline (speedup 1.0000x reference)
import functools

import jax
import jax.numpy as jnp
from jax import lax
from jax.experimental import pallas as pl
from jax.experimental.pallas import tpu as pltpu
from jax.experimental.pallas import tpu_sc as plsc

_NC = 2
_NS = 16
_NW = _NC * _NS
_L = 16
_BT = 128

_P = 5


def _body(
    h,
    d,
    clip,
    time_hbm,
    table_hbm,
    out_hbm,
    table_v,
    t_v,
    ob0,
    ob1,
    wsem0,
    wsem1,
):
    wid = lax.axis_index("s") * _NC + lax.axis_index("c")
    n_pairs = h * h
    n_chunks = n_pairs // _P
    n2 = n_chunks // 2
    n_g = _BT // _L

    pltpu.sync_copy(table_hbm, table_v)
    pltpu.sync_copy(time_hbm.at[pl.ds(wid * _BT * h, _BT * h)], t_v)

    lane = lax.iota(jnp.int32, _L)
    laneh = lane * h

    def compute(chunk, ob):
        p0 = chunk * _P
        for p_loc in range(_P):
            p = p0 + p_loc
            i = p // h
            q = p - i * h

            @plsc.parallel_loop(0, n_g, unroll=2)
            def grp(g):
                gb = g * (_L * h)
                ti = plsc.load_gather(t_v, [laneh + (gb + i)])
                tq = plsc.load_gather(t_v, [laneh + (gb + q)])
                rows16 = jnp.minimum(jnp.abs(ti - tq), clip)
                wb = rows16 * d
                for c in range(d):
                    v = plsc.load_gather(table_v, [wb + c])
                    ob[p_loc, c // 8, pl.ds((c % 8) * _BT + g * _L, _L)] = v

    def issue_write(chunk, ob, sem):
        pltpu.async_copy(
            ob, out_hbm.at[pl.ds(chunk * _P, _P), :, wid, :], sem
        )

    def wait_write(ob, sem):
        pltpu.make_async_copy(
            ob, out_hbm.at[pl.ds(0, _P), :, wid, :], sem
        ).wait()

    def body(it, carry):
        c0 = 2 * it

        @pl.when(it > 0)
        def _():
            wait_write(ob0, wsem0)

        compute(c0, ob0)
        issue_write(c0, ob0, wsem0)

        @pl.when(it > 0)
        def _():
            wait_write(ob1, wsem1)

        compute(c0 + 1, ob1)
        issue_write(c0 + 1, ob1, wsem1)
        return carry

    lax.fori_loop(0, n2, body, 0)
    wait_write(ob0, wsem0)
    wait_write(ob1, wsem1)


def kernel(time, table, max_len):
    b, h = time.shape
    v, d = table.shape
    clip = v - 1

    n_pairs = h * h
    assert b % (_NW * _BT) == 0 or b == _NW * _BT
    assert d % 8 == 0 and n_pairs % (2 * _P) == 0 and _BT % _L == 0
    nbt = b // _BT
    assert nbt == _NW
    nct = d // 8

    mesh = plsc.VectorSubcoreMesh(core_axis_name="c", subcore_axis_name="s")
    out = pl.kernel(
        functools.partial(_body, h, d, clip),
        out_type=jax.ShapeDtypeStruct((n_pairs, nct, nbt, 8 * _BT), jnp.float32),
        mesh=mesh,
        scratch_types=[
            pltpu.VMEM((v * d,), jnp.float32),
            pltpu.VMEM((_BT * h,), jnp.int32),
            pltpu.VMEM((_P, nct, 8 * _BT), jnp.float32),
            pltpu.VMEM((_P, nct, 8 * _BT), jnp.float32),
            pltpu.SemaphoreType.DMA,
            pltpu.SemaphoreType.DMA,
        ],
        compiler_params=pltpu.CompilerParams(
            use_tc_tiling_on_sc=False, needs_layout_passes=False
        ),
    )(time.reshape(b * h), table.reshape(v * d))
    phys = out.reshape(h, h, nct, nbt, 8, _BT)
    res = phys.transpose(3, 5, 0, 1, 2, 4)
    return res.reshape(b, h, h, d)


# speedup vs baseline: 1.9128x; 1.9128x over previous
import functools

import jax
import jax.numpy as jnp
from jax import lax
from jax.experimental import pallas as pl
from jax.experimental.pallas import tpu as pltpu
from jax.experimental.pallas import tpu_sc as plsc

_NC = 2
_NS = 16
_NW = _NC * _NS
_L = 16
_BT = 128

_P = 5


def _body(
    h,
    d,
    dp,
    clip,
    time_hbm,
    table_hbm,
    out_hbm,
    table_v,
    t_v,
    ob0,
    ob1,
    wsem0,
    wsem1,
):
    wid = lax.axis_index("s") * _NC + lax.axis_index("c")
    n_pairs = h * h
    n_chunks = n_pairs // _P
    n2 = n_chunks // 2
    n_g = _BT // _L

    pltpu.sync_copy(table_hbm, table_v)
    pltpu.sync_copy(time_hbm.at[pl.ds(wid * _BT * h, _BT * h)], t_v)

    lane = lax.iota(jnp.int32, _L)
    laneh = lane * h

    def compute(chunk, ob):
        p0 = chunk * _P
        for p_loc in range(_P):
            p = p0 + p_loc
            i = p // h
            q = p - i * h

            @plsc.parallel_loop(0, n_g, unroll=1)
            def grp(g):
                gb = g * (_L * h)
                ti = plsc.load_gather(t_v, [laneh + (gb + i)])
                tq = plsc.load_gather(t_v, [laneh + (gb + q)])
                rows16 = jnp.minimum(jnp.abs(ti - tq), clip)
                wb = rows16 * dp
                for c in range(d):
                    v = plsc.load_gather(table_v, [wb + c])
                    ob[p_loc, c // 8, pl.ds((c % 8) * _BT + g * _L, _L)] = v

    def issue_write(chunk, ob, sem):
        pltpu.async_copy(
            ob, out_hbm.at[pl.ds(chunk * _P, _P), :, wid, :], sem
        )

    def wait_write(ob, sem):
        pltpu.make_async_copy(
            ob, out_hbm.at[pl.ds(0, _P), :, wid, :], sem
        ).wait()

    def body(it, carry):
        c0 = 2 * it

        @pl.when(it > 0)
        def _():
            wait_write(ob0, wsem0)

        compute(c0, ob0)
        issue_write(c0, ob0, wsem0)

        @pl.when(it > 0)
        def _():
            wait_write(ob1, wsem1)

        compute(c0 + 1, ob1)
        issue_write(c0 + 1, ob1, wsem1)
        return carry

    lax.fori_loop(0, n2, body, 0)
    wait_write(ob0, wsem0)
    wait_write(ob1, wsem1)


def kernel(time, table, max_len):
    b, h = time.shape
    v, d = table.shape
    clip = v - 1

    n_pairs = h * h
    assert b % (_NW * _BT) == 0 or b == _NW * _BT
    assert d % 8 == 0 and n_pairs % (2 * _P) == 0 and _BT % _L == 0
    nbt = b // _BT
    assert nbt == _NW
    nct = d // 8

    dp = d + 1
    table_pad = jnp.concatenate(
        [table, jnp.zeros((v, 1), jnp.float32)], axis=1
    ).reshape(v * dp)

    mesh = plsc.VectorSubcoreMesh(core_axis_name="c", subcore_axis_name="s")
    out = pl.kernel(
        functools.partial(_body, h, d, dp, clip),
        out_type=jax.ShapeDtypeStruct((n_pairs, nct, nbt, 8 * _BT), jnp.float32),
        mesh=mesh,
        scratch_types=[
            pltpu.VMEM((v * dp,), jnp.float32),
            pltpu.VMEM((_BT * h,), jnp.int32),
            pltpu.VMEM((_P, nct, 8 * _BT), jnp.float32),
            pltpu.VMEM((_P, nct, 8 * _BT), jnp.float32),
            pltpu.SemaphoreType.DMA,
            pltpu.SemaphoreType.DMA,
        ],
        compiler_params=pltpu.CompilerParams(
            use_tc_tiling_on_sc=False, needs_layout_passes=False
        ),
    )(time.reshape(b * h), table_pad)
    phys = out.reshape(h, h, nct, nbt, 8, _BT)
    res = phys.transpose(3, 5, 0, 1, 2, 4)
    return res.reshape(b, h, h, d)
